# Initial kernel scaffold; baseline (speedup 1.0000x reference)
import jax
import jax.numpy as jnp
from jax.experimental import pallas as pl


def kernel(x, edge_index, W1, b1, W2, b2):
    raise NotImplementedError("write your pallas kernel here")


# speedup vs baseline: 5.0788x; 5.0788x over previous
import functools

import jax
import jax.numpy as jnp
from jax import lax
from jax.experimental import pallas as pl
from jax.experimental.pallas import tpu as pltpu
from jax.experimental.pallas import tpu_sc as plsc

N = 10000
E = 320000
D = 128
N_P = 10240

NC = 2
NS = 16
NW = NC * NS

EPT_DEG = E // NW
CH_DEG = 80
IT_DEG = EPT_DEG // CH_DEG

CH = 128
IT_AGG = 79
EPT = CH * IT_AGG
E_PAD = EPT * NW
N_PAD_ROWS = 8

RPT = N_P // NS
ZROWS = 128

_f32 = jnp.float32


def _mesh():
    return plsc.VectorSubcoreMesh(
        core_axis_name="c", subcore_axis_name="s",
        num_cores=NC, num_subcores=NS)


@functools.cache
def _sc_degrees_call():
    return functools.partial(
        pl.kernel,
        mesh=_mesh(),
        out_type=jax.ShapeDtypeStruct((NC, N_P, D), _f32),
        scratch_types=dict(
            acc=pltpu.VMEM_SHARED((N_P, D), _f32),
            idx_s=pltpu.VMEM((CH_DEG,), jnp.int32),
            idx_d=pltpu.VMEM((CH_DEG,), jnp.int32),
            rows_s=pltpu.VMEM((CH_DEG, D), _f32),
            rows_d=pltpu.VMEM((CH_DEG, D), _f32),
            zbuf=pltpu.VMEM((ZROWS, D), _f32),
        ),
    )(_sc_degrees_body)


def _sc_degrees_body(src_hbm, dst_hbm, degp_hbm, acc, idx_s, idx_d, rows_s,
                     rows_d, zbuf):
    c = lax.axis_index("c")
    s = lax.axis_index("s")
    wid = s * NC + c

    lane = lax.iota(jnp.int32, 16)
    hot0 = jnp.where(lane == 0, 1.0, 0.0).astype(_f32)
    hot1 = jnp.where(lane == 1, 1.0, 0.0).astype(_f32)
    zv = jnp.zeros((16,), _f32)

    def fill(i, _):
        rows_s[i % CH_DEG, pl.ds(0, 16)] = hot0
        rows_d[i % CH_DEG, pl.ds(0, 16)] = hot1
        for j in range(1, D // 16):
            rows_s[i % CH_DEG, pl.ds(j * 16, 16)] = zv
            rows_d[i % CH_DEG, pl.ds(j * 16, 16)] = zv
        for j in range(D // 16):
            zbuf[i, pl.ds(j * 16, 16)] = zv
        return 0

    lax.fori_loop(0, ZROWS, fill, 0)

    for k in range(RPT // ZROWS):
        pltpu.sync_copy(zbuf, acc.at[pl.ds(s * RPT + k * ZROWS, ZROWS)])
    plsc.subcore_barrier()

    def step(i, _):
        base = wid * EPT_DEG + i * CH_DEG
        pltpu.sync_copy(src_hbm.at[pl.ds(base, CH_DEG)], idx_s)
        pltpu.sync_copy(dst_hbm.at[pl.ds(base, CH_DEG)], idx_d)
        pltpu.sync_copy(rows_s, acc.at[idx_s], add=True)
        pltpu.sync_copy(rows_d, acc.at[idx_d], add=True)
        return 0

    lax.fori_loop(0, IT_DEG, step, 0)
    plsc.subcore_barrier()

    pltpu.sync_copy(acc.at[pl.ds(s * RPT, RPT)],
                    degp_hbm.at[c, pl.ds(s * RPT, RPT)])


@functools.cache
def _sc_aggregate_call():
    return functools.partial(
        pl.kernel,
        mesh=_mesh(),
        out_type=jax.ShapeDtypeStruct((NC, N_P, D), _f32),
        scratch_types=dict(
            acc=pltpu.VMEM_SHARED((N_P, D), _f32),
            idx_s=pltpu.VMEM((CH,), jnp.int32),
            idx_d=pltpu.VMEM((CH,), jnp.int32),
            rows=pltpu.VMEM((CH, D), _f32),
            zbuf=pltpu.VMEM((ZROWS, D), _f32),
            sem=pltpu.SemaphoreType.DMA,
        ),
    )(_sc_aggregate_body)


def _sc_aggregate_body(y_hbm, src_hbm, dst_hbm, part_hbm, acc, idx_s, idx_d,
                       rows, zbuf, sem):
    c = lax.axis_index("c")
    s = lax.axis_index("s")
    wid = s * NC + c

    zv = jnp.zeros((16,), _f32)

    def zfill(i, _):
        for j in range(D // 16):
            zbuf[i, pl.ds(j * 16, 16)] = zv
        return 0

    lax.fori_loop(0, ZROWS, zfill, 0)
    for k in range(RPT // ZROWS):
        pltpu.sync_copy(zbuf, acc.at[pl.ds(s * RPT + k * ZROWS, ZROWS)])
    plsc.subcore_barrier()

    def step(i, _):
        base = wid * EPT + i * CH
        pltpu.sync_copy(src_hbm.at[pl.ds(base, CH)], idx_s)
        pltpu.sync_copy(dst_hbm.at[pl.ds(base, CH)], idx_d)
        pltpu.async_copy(y_hbm.at[idx_s], rows, sem).wait()
        pltpu.sync_copy(rows, acc.at[idx_d], add=True)
        return 0

    lax.fori_loop(0, IT_AGG, step, 0)
    plsc.subcore_barrier()

    pltpu.sync_copy(acc.at[pl.ds(s * RPT, RPT)],
                    part_hbm.at[c, pl.ds(s * RPT, RPT)])


_RB = 2048
_GRID = N_P // _RB


def _mm_body(x_ref, w_ref, o_ref):
    o_ref[...] = jnp.dot(x_ref[...], w_ref[...],
                         preferred_element_type=_f32)


def _tc_matmul(x, w):
    return pl.pallas_call(
        _mm_body,
        grid=(_GRID,),
        in_specs=[
            pl.BlockSpec((_RB, D), lambda i: (i, 0)),
            pl.BlockSpec((D, D), lambda i: (0, 0)),
        ],
        out_specs=pl.BlockSpec((_RB, D), lambda i: (i, 0)),
        out_shape=jax.ShapeDtypeStruct((N_P, D), _f32),
    )(x, w)


def _norm_scale_body(y_ref, degp_ref, ys_ref, ns_ref, nd_ref):
    p = degp_ref[...]
    d = p[0] + p[1]
    d_out = d[:, 0:1]
    d_in = d[:, 1:2]
    ns = jnp.where(d_out > 0, lax.rsqrt(d_out), 0.0)
    nd = jnp.where(d_in > 0, lax.rsqrt(d_in), 0.0)
    ns_ref[...] = jnp.broadcast_to(ns, (_RB, 16))
    nd_ref[...] = jnp.broadcast_to(nd, (_RB, 16))
    ys_ref[...] = y_ref[...] * ns


def _tc_norm_scale(y, degp):
    return pl.pallas_call(
        _norm_scale_body,
        grid=(_GRID,),
        in_specs=[
            pl.BlockSpec((_RB, D), lambda i: (i, 0)),
            pl.BlockSpec((NC, _RB, D), lambda i: (0, i, 0)),
        ],
        out_specs=[
            pl.BlockSpec((_RB, D), lambda i: (i, 0)),
            pl.BlockSpec((_RB, 16), lambda i: (i, 0)),
            pl.BlockSpec((_RB, 16), lambda i: (i, 0)),
        ],
        out_shape=[
            jax.ShapeDtypeStruct((N_P, D), _f32),
            jax.ShapeDtypeStruct((N_P, 16), _f32),
            jax.ShapeDtypeStruct((N_P, 16), _f32),
        ],
    )(y, degp)


def _mid_body(p_ref, ns_ref, nd_ref, b_ref, o_ref):
    p = p_ref[...]
    t = (p[0] + p[1]) * nd_ref[...][:, :1] + b_ref[...]
    o_ref[...] = jnp.maximum(t, 0.0) * ns_ref[...][:, :1]


def _tc_mid(parts, ns, nd, b):
    return pl.pallas_call(
        _mid_body,
        grid=(_GRID,),
        in_specs=[
            pl.BlockSpec((NC, _RB, D), lambda i: (0, i, 0)),
            pl.BlockSpec((_RB, 16), lambda i: (i, 0)),
            pl.BlockSpec((_RB, 16), lambda i: (i, 0)),
            pl.BlockSpec((1, D), lambda i: (0, 0)),
        ],
        out_specs=pl.BlockSpec((_RB, D), lambda i: (i, 0)),
        out_shape=jax.ShapeDtypeStruct((N_P, D), _f32),
    )(parts, ns, nd, b)


def _final_body(p_ref, nd_ref, w_ref, b_ref, o_ref):
    p = p_ref[...]
    t = (p[0] + p[1]) * nd_ref[...][:, :1]
    o_ref[...] = jnp.dot(t, w_ref[...],
                         preferred_element_type=_f32) + b_ref[...]


_RBF = 2000


def _tc_final(parts, nd, w, b):
    return pl.pallas_call(
        _final_body,
        grid=(N // _RBF,),
        in_specs=[
            pl.BlockSpec((NC, _RBF, D), lambda i: (0, i, 0)),
            pl.BlockSpec((_RBF, 16), lambda i: (i, 0)),
            pl.BlockSpec((D, D), lambda i: (0, 0)),
            pl.BlockSpec((1, D), lambda i: (0, 0)),
        ],
        out_specs=pl.BlockSpec((_RBF, D), lambda i: (i, 0)),
        out_shape=jax.ShapeDtypeStruct((N, D), _f32),
    )(parts, nd, w, b)


def kernel(x, edge_index, W1, b1, W2, b2):
    src = edge_index[0]
    dst = edge_index[1]

    npad = E_PAD - E
    pad_ar = lax.iota(jnp.int32, npad)
    src_p = jnp.concatenate([src, N + (pad_ar % N_PAD_ROWS)])
    dst_p = jnp.concatenate([dst, pad_ar % N])

    x_p = jnp.concatenate([x, jnp.zeros((N_P - N, D), _f32)])
    b1r = b1.reshape(1, D)
    b2r = b2.reshape(1, D)

    y1 = _tc_matmul(x_p, W1)
    degp = _sc_degrees_call()(src, dst)
    y1s, ns, nd = _tc_norm_scale(y1, degp)

    p1 = _sc_aggregate_call()(y1s, src_p, dst_p)
    h1s = _tc_mid(p1, ns, nd, b1r)
    p2 = _sc_aggregate_call()(h1s, src_p, dst_p)
    return _tc_final(p2, nd, W2, b2r)
